# baseline (speedup 1.0000x reference)
import functools

import jax
import jax.numpy as jnp
from jax import lax
from jax.experimental import pallas as pl
from jax.experimental.pallas import tpu as pltpu
from jax.experimental.pallas import tpu_sc as plsc

H_RES, W_RES = 1024, 2048
N_TOTAL = 2097152
INV_2PI = 0.15915494309189535
INV_PI = 0.3183098861837907

ROWS, COLS = 4096, 512
TC_BLOCK_ROWS = 512

NUM_CORES, NUM_SUBCORES = 2, 16
NW = NUM_CORES * NUM_SUBCORES
DW = N_TOTAL // NW
CHUNK = 2048
SUB = 128
LANES = 16


_ATAN_C = (0.9999998864165668, -0.33332597213024234, 0.1998590967770574,
           -0.1416124947328797, 0.10499020128168486, -0.07235009357159948,
           0.03978298699370322, -0.014402436802774879, 0.002456994955313291)
_PI = 3.141592653589793
_HALF_PI = 1.5707963267948966


def _atan01(r):
    r2 = r * r
    p = jnp.float32(_ATAN_C[-1])
    for c in reversed(_ATAN_C[:-1]):
        p = p * r2 + jnp.float32(c)
    return p * r


def _atan2(a, b):
    aa = jnp.abs(a)
    ab = jnp.abs(b)
    m = jnp.minimum(aa, ab)
    big = jnp.maximum(aa, ab)
    r = m / jnp.maximum(big, 1e-30)
    t = _atan01(r)
    t = jnp.where(aa > ab, _HALF_PI - t, t)
    t = jnp.where(b < 0.0, _PI - t, t)
    return jnp.where(a < 0.0, -t, t)


def _uv_body(lx_ref, ly_ref, lz_ref, pk_ref, fx_ref, fy_ref):
    x = lx_ref[...]
    y = ly_ref[...]
    z = lz_ref[...]
    u = _atan2(x, -z)
    u = u * INV_2PI + 0.5
    yc = jnp.clip(y, -1.0 + 1e-6, 1.0 - 1e-6)
    s = jnp.sqrt(jnp.maximum((1.0 - yc) * (1.0 + yc), 0.0))
    v = _atan2(s, yc) * INV_PI
    u = jnp.clip(u, 0.0, 1.0)
    v = jnp.clip(v, 0.0, 1.0)
    px = u * W_RES - 0.5
    py = v * H_RES - 0.5
    px0 = jnp.floor(px)
    py0 = jnp.floor(py)
    fx_ref[...] = px - px0
    fy_ref[...] = py - py0
    xi = px0.astype(jnp.int32) & (W_RES - 1)
    yi = py0.astype(jnp.int32) & (H_RES - 1)
    pk_ref[...] = (yi << 11) | xi


def _uv_stage(lx, ly, lz):
    grid = ROWS // TC_BLOCK_ROWS
    spec = pl.BlockSpec((TC_BLOCK_ROWS, COLS), lambda i: (i, 0))
    return pl.pallas_call(
        _uv_body,
        grid=(grid,),
        in_specs=[spec, spec, spec],
        out_specs=[spec, spec,
                   pl.BlockSpec((TC_BLOCK_ROWS, COLS), lambda i: (i, 0))],
        out_shape=[
            jax.ShapeDtypeStruct((ROWS, COLS), jnp.int32),
            jax.ShapeDtypeStruct((ROWS, COLS), jnp.float32),
            jax.ShapeDtypeStruct((ROWS, COLS), jnp.float32),
        ],
    )(lx, ly, lz)


def _sc_body(pk_hbm, fx_hbm, fy_hbm, base3_hbm, out_hbm,
             pk_v, fx_v, fy_v,
             i00_v, i01_v, i10_v, i11_v,
             r00_v, r01_v, r10_v, r11_v,
             out_v, sem):
    wid = lax.axis_index("s") * NUM_CORES + lax.axis_index("c")
    lanes = lax.broadcasted_iota(jnp.int32, (LANES,), 0)
    pairs = ((i00_v, r00_v), (i01_v, r01_v), (i10_v, r10_v), (i11_v, r11_v))

    def chunk_body(ci, _):
        off = wid * DW + ci * CHUNK
        pltpu.sync_copy(pk_hbm.at[pl.ds(off, CHUNK)], pk_v)
        pltpu.sync_copy(fx_hbm.at[pl.ds(off, CHUNK)], fx_v)
        pltpu.sync_copy(fy_hbm.at[pl.ds(off, CHUNK)], fy_v)

        def idx_body(j, _):
            s = j * LANES
            pk16 = pk_v[pl.ds(s, LANES)]
            x0 = pk16 & (W_RES - 1)
            y0 = pk16 >> 11
            x1 = (x0 + 1) & (W_RES - 1)
            y1 = (y0 + 1) & (H_RES - 1)
            row1 = y1 << 11
            i00_v[pl.ds(s, LANES)] = pk16
            i01_v[pl.ds(s, LANES)] = pk16 - x0 + x1
            i10_v[pl.ds(s, LANES)] = row1 | x0
            i11_v[pl.ds(s, LANES)] = row1 | x1
            return 0
        lax.fori_loop(0, CHUNK // LANES, idx_body, 0)

        for iv, rv in pairs:
            pltpu.make_async_copy(base3_hbm.at[iv], rv, sem).start()
        for iv, rv in pairs:
            pltpu.make_async_copy(base3_hbm.at[iv], rv, sem).wait()

        def c_body(j, _):
            s = j * LANES
            dvec = lanes + s
            fx16 = fx_v[pl.ds(s, LANES)]
            fy16 = fy_v[pl.ds(s, LANES)]
            gx = 1.0 - fx16
            gy = 1.0 - fy16
            w00 = gx * gy
            w01 = fx16 * gy
            w10 = gx * fy16
            w11 = fx16 * fy16
            for c in range(3):
                cc = jnp.full((LANES,), c, jnp.int32)
                t00 = plsc.load_gather(r00_v, [dvec, cc])
                t01 = plsc.load_gather(r01_v, [dvec, cc])
                t10 = plsc.load_gather(r10_v, [dvec, cc])
                t11 = plsc.load_gather(r11_v, [dvec, cc])
                o = jnp.exp(w00 * t00 + w01 * t01 + w10 * t10 + w11 * t11)
                plsc.store_scatter(out_v, [dvec, cc], o)
            return 0
        lax.fori_loop(0, CHUNK // LANES, c_body, 0)

        pltpu.sync_copy(out_v, out_hbm.at[pl.ds(off, CHUNK)])
        return 0

    lax.fori_loop(0, DW // CHUNK, chunk_body, 0)


_sc_stage = functools.partial(
    pl.kernel,
    out_type=jax.ShapeDtypeStruct((N_TOTAL, 3), jnp.float32),
    mesh=plsc.VectorSubcoreMesh(core_axis_name="c", subcore_axis_name="s"),
    compiler_params=pltpu.CompilerParams(needs_layout_passes=False,
                                         use_tc_tiling_on_sc=False),
    scratch_types=[
        pltpu.VMEM((CHUNK,), jnp.int32),
        pltpu.VMEM((CHUNK,), jnp.float32),
        pltpu.VMEM((CHUNK,), jnp.float32),
        pltpu.VMEM((CHUNK,), jnp.int32),
        pltpu.VMEM((CHUNK,), jnp.int32),
        pltpu.VMEM((CHUNK,), jnp.int32),
        pltpu.VMEM((CHUNK,), jnp.int32),
        pltpu.VMEM((CHUNK, 8), jnp.float32),
        pltpu.VMEM((CHUNK, 8), jnp.float32),
        pltpu.VMEM((CHUNK, 8), jnp.float32),
        pltpu.VMEM((CHUNK, 8), jnp.float32),
        pltpu.VMEM((CHUNK, 3), jnp.float32),
        pltpu.SemaphoreType.DMA,
    ],
)(_sc_body)


def kernel(l, base):
    lx = l[:, 0].reshape(ROWS, COLS)
    ly = l[:, 1].reshape(ROWS, COLS)
    lz = l[:, 2].reshape(ROWS, COLS)
    pk, fx, fy = _uv_stage(lx, ly, lz)
    base3 = jnp.pad(base.reshape(H_RES * W_RES, 3), ((0, 0), (0, 5)))
    return _sc_stage(pk.reshape(N_TOTAL), fx.reshape(N_TOTAL),
                     fy.reshape(N_TOTAL), base3)


# speedup vs baseline: 9.6365x; 9.6365x over previous
import functools

import jax
import jax.numpy as jnp
from jax import lax
from jax.experimental import pallas as pl
from jax.experimental.pallas import tpu as pltpu
from jax.experimental.pallas import tpu_sc as plsc

H_RES, W_RES = 1024, 2048
HW = H_RES * W_RES
N_TOTAL = 2097152
INV_2PI = 0.15915494309189535
INV_PI = 0.3183098861837907

ROWS, COLS = 16384, 128
TC_BLOCK_ROWS = 2048

NUM_CORES, NUM_SUBCORES = 2, 16
NW = NUM_CORES * NUM_SUBCORES
DW = N_TOTAL // NW
CHUNK = 2048
TW = HW // NW
TCHUNK = 2048
LANES = 16


_ATAN_C = (0.9999998864165668, -0.33332597213024234, 0.1998590967770574,
           -0.1416124947328797, 0.10499020128168486, -0.07235009357159948,
           0.03978298699370322, -0.014402436802774879, 0.002456994955313291)
_PI = 3.141592653589793
_HALF_PI = 1.5707963267948966


def _atan01(r):
    r2 = r * r
    p = jnp.float32(_ATAN_C[-1])
    for c in reversed(_ATAN_C[:-1]):
        p = p * r2 + jnp.float32(c)
    return p * r


def _atan2(a, b):
    aa = jnp.abs(a)
    ab = jnp.abs(b)
    m = jnp.minimum(aa, ab)
    big = jnp.maximum(aa, ab)
    r = m / jnp.maximum(big, 1e-30)
    t = _atan01(r)
    t = jnp.where(aa > ab, _HALF_PI - t, t)
    t = jnp.where(b < 0.0, _PI - t, t)
    return jnp.where(a < 0.0, -t, t)


def _uv_body(lx_ref, ly_ref, lz_ref, pk_ref, fx_ref, fy_ref):
    x = lx_ref[...]
    y = ly_ref[...]
    z = lz_ref[...]
    u = _atan2(x, -z)
    u = u * INV_2PI + 0.5
    yc = jnp.clip(y, -1.0 + 1e-6, 1.0 - 1e-6)
    s = jnp.sqrt(jnp.maximum((1.0 - yc) * (1.0 + yc), 0.0))
    v = _atan2(s, yc) * INV_PI
    u = jnp.clip(u, 0.0, 1.0)
    v = jnp.clip(v, 0.0, 1.0)
    px = u * W_RES - 0.5
    py = v * H_RES - 0.5
    px0 = jnp.floor(px)
    py0 = jnp.floor(py)
    fx_ref[...] = px - px0
    fy_ref[...] = py - py0
    xi = px0.astype(jnp.int32) & (W_RES - 1)
    yi = py0.astype(jnp.int32) & (H_RES - 1)
    pk_ref[...] = (yi << 11) | xi


def _uv_stage(lx, ly, lz):
    grid = ROWS // TC_BLOCK_ROWS
    spec = pl.BlockSpec((TC_BLOCK_ROWS, COLS), lambda i: (i, 0))
    return pl.pallas_call(
        _uv_body,
        grid=(grid,),
        in_specs=[spec, spec, spec],
        out_specs=[spec, spec, spec],
        out_shape=[
            jax.ShapeDtypeStruct((ROWS, COLS), jnp.int32),
            jax.ShapeDtypeStruct((ROWS, COLS), jnp.float32),
            jax.ShapeDtypeStruct((ROWS, COLS), jnp.float32),
        ],
    )(lx, ly, lz)


def _detile_body(p0_ref, p1_ref, p2_ref, o0_ref, o1_ref, o2_ref):
    o0_ref[:, 0, 0, :] = p0_ref[...]
    o1_ref[:, 0, 0, :] = p1_ref[...]
    o2_ref[:, 0, 0, :] = p2_ref[...]


def _detile_stage(p0, p1, p2):
    in_spec = pl.BlockSpec((512, 128), lambda i, j: (i, j))
    out_spec = pl.BlockSpec((512, 1, 1, 128), lambda i, j: (i, j, 0, 0))
    return pl.pallas_call(
        _detile_body,
        grid=(H_RES // 512, W_RES // 128),
        in_specs=[in_spec, in_spec, in_spec],
        out_specs=[out_spec, out_spec, out_spec],
        out_shape=[jax.ShapeDtypeStruct((H_RES, W_RES // 128, 1, 128),
                                        jnp.float32)] * 3,
    )(p0, p1, p2)


def _table_body(pr_hbm, pg_hbm, pb_hbm, tab_hbm, pr_v, pg_v, pb_v, tab_v):
    wid = lax.axis_index("s") * NUM_CORES + lax.axis_index("c")
    lanes = lax.broadcasted_iota(jnp.int32, (LANES,), 0)
    c0 = jnp.full((LANES,), 0, jnp.int32)
    c1 = jnp.full((LANES,), 1, jnp.int32)
    c2 = jnp.full((LANES,), 2, jnp.int32)

    def chunk_body(ci, _):
        off = wid * TW + ci * TCHUNK
        pltpu.sync_copy(pr_hbm.at[pl.ds(off, TCHUNK)], pr_v)
        pltpu.sync_copy(pg_hbm.at[pl.ds(off, TCHUNK)], pg_v)
        pltpu.sync_copy(pb_hbm.at[pl.ds(off, TCHUNK)], pb_v)

        def t_body(j, _):
            s = j * LANES
            tvec = lanes + s
            plsc.store_scatter(tab_v, [tvec, c0], pr_v[pl.ds(s, LANES)])
            plsc.store_scatter(tab_v, [tvec, c1], pg_v[pl.ds(s, LANES)])
            plsc.store_scatter(tab_v, [tvec, c2], pb_v[pl.ds(s, LANES)])
            return 0
        lax.fori_loop(0, TCHUNK // LANES, t_body, 0)

        pltpu.sync_copy(tab_v, tab_hbm.at[pl.ds(off, TCHUNK)])
        return 0

    lax.fori_loop(0, TW // TCHUNK, chunk_body, 0)


_table_stage = functools.partial(
    pl.kernel,
    out_type=jax.ShapeDtypeStruct((HW, 8), jnp.float32),
    mesh=plsc.VectorSubcoreMesh(core_axis_name="c", subcore_axis_name="s"),
    compiler_params=pltpu.CompilerParams(needs_layout_passes=False,
                                         use_tc_tiling_on_sc=False),
    scratch_types=[
        pltpu.VMEM((TCHUNK,), jnp.float32),
        pltpu.VMEM((TCHUNK,), jnp.float32),
        pltpu.VMEM((TCHUNK,), jnp.float32),
        pltpu.VMEM((TCHUNK, 8), jnp.float32),
    ],
)(_table_body)


def _sc_body(pk_hbm, fx_hbm, fy_hbm, base3_hbm,
             or_hbm, og_hbm, ob_hbm,
             pk_v, fx_v, fy_v,
             i00_v, i01_v, i10_v, i11_v,
             r00_v, r01_v, r10_v, r11_v,
             o0_v, o1_v, o2_v, sem):
    wid = lax.axis_index("s") * NUM_CORES + lax.axis_index("c")
    lanes = lax.broadcasted_iota(jnp.int32, (LANES,), 0)
    pairs = ((i00_v, r00_v), (i01_v, r01_v), (i10_v, r10_v), (i11_v, r11_v))

    def chunk_body(ci, _):
        off = wid * DW + ci * CHUNK
        pltpu.sync_copy(pk_hbm.at[pl.ds(off, CHUNK)], pk_v)
        pltpu.sync_copy(fx_hbm.at[pl.ds(off, CHUNK)], fx_v)
        pltpu.sync_copy(fy_hbm.at[pl.ds(off, CHUNK)], fy_v)

        def idx_body(j, _):
            s = j * LANES
            pk16 = pk_v[pl.ds(s, LANES)]
            x0 = pk16 & (W_RES - 1)
            y0 = pk16 >> 11
            x1 = (x0 + 1) & (W_RES - 1)
            y1 = (y0 + 1) & (H_RES - 1)
            row1 = y1 << 11
            i00_v[pl.ds(s, LANES)] = pk16
            i01_v[pl.ds(s, LANES)] = pk16 - x0 + x1
            i10_v[pl.ds(s, LANES)] = row1 | x0
            i11_v[pl.ds(s, LANES)] = row1 | x1
            return 0
        lax.fori_loop(0, CHUNK // LANES, idx_body, 0)

        for iv, rv in pairs:
            pltpu.make_async_copy(base3_hbm.at[iv], rv, sem).start()
        for iv, rv in pairs:
            pltpu.make_async_copy(base3_hbm.at[iv], rv, sem).wait()

        outs = (o0_v, o1_v, o2_v)

        def c_body(j, _):
            s = j * LANES
            dvec = lanes + s
            fx16 = fx_v[pl.ds(s, LANES)]
            fy16 = fy_v[pl.ds(s, LANES)]
            gx = 1.0 - fx16
            gy = 1.0 - fy16
            w00 = gx * gy
            w01 = fx16 * gy
            w10 = gx * fy16
            w11 = fx16 * fy16
            for c in range(3):
                cc = jnp.full((LANES,), c, jnp.int32)
                t00 = plsc.load_gather(r00_v, [dvec, cc])
                t01 = plsc.load_gather(r01_v, [dvec, cc])
                t10 = plsc.load_gather(r10_v, [dvec, cc])
                t11 = plsc.load_gather(r11_v, [dvec, cc])
                o = jnp.exp(w00 * t00 + w01 * t01 + w10 * t10 + w11 * t11)
                outs[c][pl.ds(s, LANES)] = o
            return 0
        lax.fori_loop(0, CHUNK // LANES, c_body, 0)

        pltpu.sync_copy(o0_v, or_hbm.at[pl.ds(off, CHUNK)])
        pltpu.sync_copy(o1_v, og_hbm.at[pl.ds(off, CHUNK)])
        pltpu.sync_copy(o2_v, ob_hbm.at[pl.ds(off, CHUNK)])
        return 0

    lax.fori_loop(0, DW // CHUNK, chunk_body, 0)


_sc_stage = functools.partial(
    pl.kernel,
    out_type=[jax.ShapeDtypeStruct((N_TOTAL,), jnp.float32)] * 3,
    mesh=plsc.VectorSubcoreMesh(core_axis_name="c", subcore_axis_name="s"),
    compiler_params=pltpu.CompilerParams(needs_layout_passes=False,
                                         use_tc_tiling_on_sc=False),
    scratch_types=[
        pltpu.VMEM((CHUNK,), jnp.int32),
        pltpu.VMEM((CHUNK,), jnp.float32),
        pltpu.VMEM((CHUNK,), jnp.float32),
        pltpu.VMEM((CHUNK,), jnp.int32),
        pltpu.VMEM((CHUNK,), jnp.int32),
        pltpu.VMEM((CHUNK,), jnp.int32),
        pltpu.VMEM((CHUNK,), jnp.int32),
        pltpu.VMEM((CHUNK, 8), jnp.float32),
        pltpu.VMEM((CHUNK, 8), jnp.float32),
        pltpu.VMEM((CHUNK, 8), jnp.float32),
        pltpu.VMEM((CHUNK, 8), jnp.float32),
        pltpu.VMEM((CHUNK,), jnp.float32),
        pltpu.VMEM((CHUNK,), jnp.float32),
        pltpu.VMEM((CHUNK,), jnp.float32),
        pltpu.SemaphoreType.DMA,
    ],
)(_sc_body)


def kernel(l, base):
    lx = l[:, 0].reshape(ROWS, COLS)
    ly = l[:, 1].reshape(ROWS, COLS)
    lz = l[:, 2].reshape(ROWS, COLS)
    pk, fx, fy = _uv_stage(lx, ly, lz)
    p0, p1, p2 = _detile_stage(base[:, :, 0], base[:, :, 1], base[:, :, 2])
    table = _table_stage(p0.reshape(HW), p1.reshape(HW), p2.reshape(HW))
    outr, outg, outb = _sc_stage(pk.reshape(N_TOTAL), fx.reshape(N_TOTAL),
                                 fy.reshape(N_TOTAL), table)
    return jnp.stack([outr, outg, outb], axis=1)


# speedup vs baseline: 24.9273x; 2.5868x over previous
import functools

import jax
import jax.numpy as jnp
from jax import lax
from jax.experimental import pallas as pl
from jax.experimental.pallas import tpu as pltpu
from jax.experimental.pallas import tpu_sc as plsc

H_RES, W_RES = 1024, 2048
HW = H_RES * W_RES
N_TOTAL = 2097152
INV_2PI = 0.15915494309189535
INV_PI = 0.3183098861837907

ROWS, COLS = 16384, 128
TC_BLOCK_ROWS = 2048

NUM_CORES, NUM_SUBCORES = 2, 16
NW = NUM_CORES * NUM_SUBCORES
DW = N_TOTAL // NW
CHUNK = 2048
TW = HW // NW
TCHUNK = 2048
LANES = 16


_ATAN_C = (0.9999998864165668, -0.33332597213024234, 0.1998590967770574,
           -0.1416124947328797, 0.10499020128168486, -0.07235009357159948,
           0.03978298699370322, -0.014402436802774879, 0.002456994955313291)
_PI = 3.141592653589793
_HALF_PI = 1.5707963267948966


def _atan01(r):
    r2 = r * r
    p = jnp.float32(_ATAN_C[-1])
    for c in reversed(_ATAN_C[:-1]):
        p = p * r2 + jnp.float32(c)
    return p * r


def _atan2(a, b):
    aa = jnp.abs(a)
    ab = jnp.abs(b)
    m = jnp.minimum(aa, ab)
    big = jnp.maximum(aa, ab)
    r = m / jnp.maximum(big, 1e-30)
    t = _atan01(r)
    t = jnp.where(aa > ab, _HALF_PI - t, t)
    t = jnp.where(b < 0.0, _PI - t, t)
    return jnp.where(a < 0.0, -t, t)


def _uv_body(lx_ref, ly_ref, lz_ref, pk_ref, fx_ref, fy_ref):
    x = lx_ref[...]
    y = ly_ref[...]
    z = lz_ref[...]
    u = _atan2(x, -z)
    u = u * INV_2PI + 0.5
    yc = jnp.clip(y, -1.0 + 1e-6, 1.0 - 1e-6)
    s = jnp.sqrt(jnp.maximum((1.0 - yc) * (1.0 + yc), 0.0))
    v = _atan2(s, yc) * INV_PI
    u = jnp.clip(u, 0.0, 1.0)
    v = jnp.clip(v, 0.0, 1.0)
    px = u * W_RES - 0.5
    py = v * H_RES - 0.5
    px0 = jnp.floor(px)
    py0 = jnp.floor(py)
    fx_ref[...] = px - px0
    fy_ref[...] = py - py0
    xi = px0.astype(jnp.int32) & (W_RES - 1)
    yi = py0.astype(jnp.int32) & (H_RES - 1)
    pk_ref[...] = (yi << 11) | xi


def _uv_stage(lx, ly, lz):
    grid = ROWS // TC_BLOCK_ROWS
    spec = pl.BlockSpec((TC_BLOCK_ROWS, COLS), lambda i: (i, 0))
    return pl.pallas_call(
        _uv_body,
        grid=(grid,),
        in_specs=[spec, spec, spec],
        out_specs=[spec, spec, spec],
        out_shape=[
            jax.ShapeDtypeStruct((ROWS, COLS), jnp.int32),
            jax.ShapeDtypeStruct((ROWS, COLS), jnp.float32),
            jax.ShapeDtypeStruct((ROWS, COLS), jnp.float32),
        ],
    )(lx, ly, lz)


def _detile_body(p0_ref, p1_ref, p2_ref, o0_ref, o1_ref, o2_ref):
    o0_ref[:, 0, 0, :] = p0_ref[...]
    o1_ref[:, 0, 0, :] = p1_ref[...]
    o2_ref[:, 0, 0, :] = p2_ref[...]


def _detile_stage(p0, p1, p2):
    in_spec = pl.BlockSpec((512, 128), lambda i, j: (i, j))
    out_spec = pl.BlockSpec((512, 1, 1, 128), lambda i, j: (i, j, 0, 0))
    return pl.pallas_call(
        _detile_body,
        grid=(H_RES // 512, W_RES // 128),
        in_specs=[in_spec, in_spec, in_spec],
        out_specs=[out_spec, out_spec, out_spec],
        out_shape=[jax.ShapeDtypeStruct((H_RES, W_RES // 128, 1, 128),
                                        jnp.float32)] * 3,
    )(p0, p1, p2)


def _table_body(pr_hbm, pg_hbm, pb_hbm, tab_hbm,
                r0_v, g0_v, b0_v, r1_v, g1_v, b1_v, tab_v, sem):
    wid = lax.axis_index("s") * NUM_CORES + lax.axis_index("c")
    lanes = lax.broadcasted_iota(jnp.int32, (LANES,), 0)
    cw = [jnp.full((LANES,), k, jnp.int32) for k in range(6)]

    def chunk_body(ci, _):
        y = wid * (H_RES // NW) + ci
        o0 = y * W_RES
        o1 = (jnp.bitwise_and(y + 1, H_RES - 1)) * W_RES
        copies = []
        for src_hbm, d0, d1 in ((pr_hbm, r0_v, r1_v), (pg_hbm, g0_v, g1_v),
                                (pb_hbm, b0_v, b1_v)):
            copies.append(pltpu.make_async_copy(
                src_hbm.at[pl.ds(o0, W_RES)], d0.at[pl.ds(0, W_RES)], sem))
            copies.append(pltpu.make_async_copy(
                src_hbm.at[pl.ds(o0, 8)], d0.at[pl.ds(W_RES, 8)], sem))
            copies.append(pltpu.make_async_copy(
                src_hbm.at[pl.ds(o1, W_RES)], d1.at[pl.ds(0, W_RES)], sem))
            copies.append(pltpu.make_async_copy(
                src_hbm.at[pl.ds(o1, 8)], d1.at[pl.ds(W_RES, 8)], sem))
        for c in copies:
            c.start()
        for c in copies:
            c.wait()

        def t_body(j, _):
            s = j * LANES
            tvec = lanes + s
            tvec1 = tvec + 1
            r00 = r0_v[pl.ds(s, LANES)]
            g00 = g0_v[pl.ds(s, LANES)]
            b00 = b0_v[pl.ds(s, LANES)]
            r10 = r1_v[pl.ds(s, LANES)]
            g10 = g1_v[pl.ds(s, LANES)]
            b10 = b1_v[pl.ds(s, LANES)]
            r01 = plsc.load_gather(r0_v, [tvec1])
            g01 = plsc.load_gather(g0_v, [tvec1])
            b01 = plsc.load_gather(b0_v, [tvec1])
            r11 = plsc.load_gather(r1_v, [tvec1])
            g11 = plsc.load_gather(g1_v, [tvec1])
            b11 = plsc.load_gather(b1_v, [tvec1])
            I = plsc.PackFormat.INTERLEAVED
            ws = (plsc.pack(r00, g00, format=I), plsc.pack(b00, r01, format=I),
                  plsc.pack(g01, b01, format=I), plsc.pack(r10, g10, format=I),
                  plsc.pack(b10, r11, format=I), plsc.pack(g11, b11, format=I))
            for k in range(6):
                plsc.store_scatter(tab_v, [tvec, cw[k]],
                                   plsc.bitcast(ws[k], jnp.float32))
            return 0
        lax.fori_loop(0, W_RES // LANES, t_body, 0)

        pltpu.sync_copy(tab_v, tab_hbm.at[pl.ds(o0, W_RES)])
        return 0

    lax.fori_loop(0, H_RES // NW, chunk_body, 0)


_table_stage = functools.partial(
    pl.kernel,
    out_type=jax.ShapeDtypeStruct((HW, 8), jnp.float32),
    mesh=plsc.VectorSubcoreMesh(core_axis_name="c", subcore_axis_name="s"),
    compiler_params=pltpu.CompilerParams(needs_layout_passes=False,
                                         use_tc_tiling_on_sc=False),
    scratch_types=[
        pltpu.VMEM((W_RES + 8,), jnp.float32),
        pltpu.VMEM((W_RES + 8,), jnp.float32),
        pltpu.VMEM((W_RES + 8,), jnp.float32),
        pltpu.VMEM((W_RES + 8,), jnp.float32),
        pltpu.VMEM((W_RES + 8,), jnp.float32),
        pltpu.VMEM((W_RES + 8,), jnp.float32),
        pltpu.VMEM((W_RES, 8), jnp.float32),
        pltpu.SemaphoreType.DMA,
    ],
)(_table_body)


NCHUNKS = DW // CHUNK


def _sc_body(pk_hbm, fx_hbm, fy_hbm, tab_hbm,
             or_hbm, og_hbm, ob_hbm,
             pk_v, fx_v, fy_v, rows_v, o0_v, o1_v, o2_v,
             semi0, semi1, semg0, semg1, semo0, semo1):
    wid = lax.axis_index("s") * NUM_CORES + lax.axis_index("c")
    lanes = lax.broadcasted_iota(jnp.int32, (LANES,), 0)
    cw = [jnp.full((LANES,), k, jnp.int32) for k in range(6)]
    base = wid * DW
    semi = (semi0, semi1)
    semg = (semg0, semg1)
    semo = (semo0, semo1)

    def in_copies(c):
        bs = c % 2
        off = base + c * CHUNK
        return [pltpu.make_async_copy(h.at[pl.ds(off, CHUNK)],
                                      v.at[bs], semi[bs])
                for h, v in ((pk_hbm, pk_v), (fx_hbm, fx_v), (fy_hbm, fy_v))]

    def gather_copy(c):
        bs = c % 2
        return pltpu.make_async_copy(tab_hbm.at[pk_v.at[bs]],
                                     rows_v.at[bs], semg[bs])

    def out_copies(c):
        bs = c % 2
        off = base + c * CHUNK
        return [pltpu.make_async_copy(v.at[bs], h.at[pl.ds(off, CHUNK)],
                                      semo[bs])
                for v, h in ((o0_v, or_hbm), (o1_v, og_hbm), (o2_v, ob_hbm))]

    def s1(c):
        for cp in in_copies(c):
            cp.start()

    def s2(c):
        for cp in in_copies(c):
            cp.wait()
        gather_copy(c).start()

    def s3(c):
        bs = c % 2
        gather_copy(c).wait()
        if c >= 2:
            for cp in out_copies(c - 2):
                cp.wait()

        def c_body(j, _):
            s = j * LANES
            dvec = lanes + s
            fx16 = fx_v[bs, pl.ds(s, LANES)]
            fy16 = fy_v[bs, pl.ds(s, LANES)]
            gx = 1.0 - fx16
            gy = 1.0 - fy16
            w00 = gx * gy
            w01 = fx16 * gy
            w10 = gx * fy16
            w11 = fx16 * fy16
            I = plsc.PackFormat.INTERLEAVED
            rbs = rows_v.at[bs]
            wv = [plsc.bitcast(plsc.load_gather(rbs, [dvec, cw[k]]),
                               jnp.bfloat16) for k in range(6)]
            r00, g00 = plsc.unpack(wv[0], format=I)
            b00, r01 = plsc.unpack(wv[1], format=I)
            g01, b01 = plsc.unpack(wv[2], format=I)
            r10, g10 = plsc.unpack(wv[3], format=I)
            b10, r11 = plsc.unpack(wv[4], format=I)
            g11, b11 = plsc.unpack(wv[5], format=I)
            o0_v[bs, pl.ds(s, LANES)] = jnp.exp(
                w00 * r00 + w01 * r01 + w10 * r10 + w11 * r11)
            o1_v[bs, pl.ds(s, LANES)] = jnp.exp(
                w00 * g00 + w01 * g01 + w10 * g10 + w11 * g11)
            o2_v[bs, pl.ds(s, LANES)] = jnp.exp(
                w00 * b00 + w01 * b01 + w10 * b10 + w11 * b11)
            return 0
        lax.fori_loop(0, CHUNK // LANES, c_body, 0)

        for cp in out_copies(c):
            cp.start()

    s1(0)
    s1(1)
    s2(0)
    for c in range(NCHUNKS):
        if c + 1 < NCHUNKS:
            s2(c + 1)
        s3(c)
        if c + 2 < NCHUNKS:
            s1(c + 2)
    for cp in out_copies(NCHUNKS - 2):
        cp.wait()
    for cp in out_copies(NCHUNKS - 1):
        cp.wait()


_sc_stage = functools.partial(
    pl.kernel,
    out_type=[jax.ShapeDtypeStruct((N_TOTAL,), jnp.float32)] * 3,
    mesh=plsc.VectorSubcoreMesh(core_axis_name="c", subcore_axis_name="s"),
    compiler_params=pltpu.CompilerParams(needs_layout_passes=False,
                                         use_tc_tiling_on_sc=False),
    scratch_types=[
        pltpu.VMEM((2, CHUNK), jnp.int32),
        pltpu.VMEM((2, CHUNK), jnp.float32),
        pltpu.VMEM((2, CHUNK), jnp.float32),
        pltpu.VMEM((2, CHUNK, 8), jnp.float32),
        pltpu.VMEM((2, CHUNK), jnp.float32),
        pltpu.VMEM((2, CHUNK), jnp.float32),
        pltpu.VMEM((2, CHUNK), jnp.float32),
        pltpu.SemaphoreType.DMA,
        pltpu.SemaphoreType.DMA,
        pltpu.SemaphoreType.DMA,
        pltpu.SemaphoreType.DMA,
        pltpu.SemaphoreType.DMA,
        pltpu.SemaphoreType.DMA,
    ],
)(_sc_body)


def kernel(l, base):
    lx = l[:, 0].reshape(ROWS, COLS)
    ly = l[:, 1].reshape(ROWS, COLS)
    lz = l[:, 2].reshape(ROWS, COLS)
    pk, fx, fy = _uv_stage(lx, ly, lz)
    p0, p1, p2 = _detile_stage(base[:, :, 0], base[:, :, 1], base[:, :, 2])
    table = _table_stage(p0.reshape(HW), p1.reshape(HW), p2.reshape(HW))
    outr, outg, outb = _sc_stage(pk.reshape(N_TOTAL), fx.reshape(N_TOTAL),
                                 fy.reshape(N_TOTAL), table)
    return jnp.stack([outr, outg, outb], axis=1)


# speedup vs baseline: 26.2875x; 1.0546x over previous
import functools

import jax
import jax.numpy as jnp
from jax import lax
from jax.experimental import pallas as pl
from jax.experimental.pallas import tpu as pltpu
from jax.experimental.pallas import tpu_sc as plsc

H_RES, W_RES = 1024, 2048
HW = H_RES * W_RES
N_TOTAL = 2097152
INV_2PI = 0.15915494309189535
INV_PI = 0.3183098861837907

ROWS, COLS = 16384, 128
TC_BLOCK_ROWS = 2048

NUM_CORES, NUM_SUBCORES = 2, 16
NW = NUM_CORES * NUM_SUBCORES
DW = N_TOTAL // NW
CHUNK = 4096
TW = HW // NW
TCHUNK = 2048
LANES = 16


_ATAN_C = (0.9999998864165668, -0.33332597213024234, 0.1998590967770574,
           -0.1416124947328797, 0.10499020128168486, -0.07235009357159948,
           0.03978298699370322, -0.014402436802774879, 0.002456994955313291)
_PI = 3.141592653589793
_HALF_PI = 1.5707963267948966


def _atan01(r):
    r2 = r * r
    p = jnp.float32(_ATAN_C[-1])
    for c in reversed(_ATAN_C[:-1]):
        p = p * r2 + jnp.float32(c)
    return p * r


def _atan2(a, b):
    aa = jnp.abs(a)
    ab = jnp.abs(b)
    m = jnp.minimum(aa, ab)
    big = jnp.maximum(aa, ab)
    r = m / jnp.maximum(big, 1e-30)
    t = _atan01(r)
    t = jnp.where(aa > ab, _HALF_PI - t, t)
    t = jnp.where(b < 0.0, _PI - t, t)
    return jnp.where(a < 0.0, -t, t)


def _uv_body(lx_ref, ly_ref, lz_ref, pk_ref, fx_ref, fy_ref):
    x = lx_ref[...]
    y = ly_ref[...]
    z = lz_ref[...]
    u = _atan2(x, -z)
    u = u * INV_2PI + 0.5
    yc = jnp.clip(y, -1.0 + 1e-6, 1.0 - 1e-6)
    s = jnp.sqrt(jnp.maximum((1.0 - yc) * (1.0 + yc), 0.0))
    v = _atan2(s, yc) * INV_PI
    u = jnp.clip(u, 0.0, 1.0)
    v = jnp.clip(v, 0.0, 1.0)
    px = u * W_RES - 0.5
    py = v * H_RES - 0.5
    px0 = jnp.floor(px)
    py0 = jnp.floor(py)
    fx_ref[...] = px - px0
    fy_ref[...] = py - py0
    xi = px0.astype(jnp.int32) & (W_RES - 1)
    yi = py0.astype(jnp.int32) & (H_RES - 1)
    pk_ref[...] = (yi << 11) | xi


def _uv_stage(lx, ly, lz):
    grid = ROWS // TC_BLOCK_ROWS
    spec = pl.BlockSpec((TC_BLOCK_ROWS, COLS), lambda i: (i, 0))
    return pl.pallas_call(
        _uv_body,
        grid=(grid,),
        in_specs=[spec, spec, spec],
        out_specs=[spec, spec, spec],
        out_shape=[
            jax.ShapeDtypeStruct((ROWS, COLS), jnp.int32),
            jax.ShapeDtypeStruct((ROWS, COLS), jnp.float32),
            jax.ShapeDtypeStruct((ROWS, COLS), jnp.float32),
        ],
    )(lx, ly, lz)


def _detile_body(p0_ref, p1_ref, p2_ref, o0_ref, o1_ref, o2_ref):
    o0_ref[:, 0, 0, :] = p0_ref[...]
    o1_ref[:, 0, 0, :] = p1_ref[...]
    o2_ref[:, 0, 0, :] = p2_ref[...]


def _detile_stage(p0, p1, p2):
    in_spec = pl.BlockSpec((1024, 128), lambda i, j: (i, j))
    out_spec = pl.BlockSpec((1024, 1, 1, 128), lambda i, j: (i, j, 0, 0))
    return pl.pallas_call(
        _detile_body,
        grid=(H_RES // 1024, W_RES // 128),
        in_specs=[in_spec, in_spec, in_spec],
        out_specs=[out_spec, out_spec, out_spec],
        out_shape=[jax.ShapeDtypeStruct((H_RES, W_RES // 128, 1, 128),
                                        jnp.float32)] * 3,
    )(p0, p1, p2)


def _table_body(pr_hbm, pg_hbm, pb_hbm, tab_hbm,
                r0_v, g0_v, b0_v, r1_v, g1_v, b1_v, tab_v, sem):
    wid = lax.axis_index("s") * NUM_CORES + lax.axis_index("c")
    lanes = lax.broadcasted_iota(jnp.int32, (LANES,), 0)
    cw = [jnp.full((LANES,), k, jnp.int32) for k in range(6)]

    def chunk_body(ci, _):
        y = wid * (H_RES // NW) + ci
        o0 = y * W_RES
        o1 = (jnp.bitwise_and(y + 1, H_RES - 1)) * W_RES
        copies = []
        for src_hbm, d0, d1 in ((pr_hbm, r0_v, r1_v), (pg_hbm, g0_v, g1_v),
                                (pb_hbm, b0_v, b1_v)):
            copies.append(pltpu.make_async_copy(
                src_hbm.at[pl.ds(o0, W_RES)], d0.at[pl.ds(0, W_RES)], sem))
            copies.append(pltpu.make_async_copy(
                src_hbm.at[pl.ds(o0, 8)], d0.at[pl.ds(W_RES, 8)], sem))
            copies.append(pltpu.make_async_copy(
                src_hbm.at[pl.ds(o1, W_RES)], d1.at[pl.ds(0, W_RES)], sem))
            copies.append(pltpu.make_async_copy(
                src_hbm.at[pl.ds(o1, 8)], d1.at[pl.ds(W_RES, 8)], sem))
        for c in copies:
            c.start()
        for c in copies:
            c.wait()

        def t_body(j, _):
            s = j * LANES
            tvec = lanes + s
            tvec1 = tvec + 1
            r00 = r0_v[pl.ds(s, LANES)]
            g00 = g0_v[pl.ds(s, LANES)]
            b00 = b0_v[pl.ds(s, LANES)]
            r10 = r1_v[pl.ds(s, LANES)]
            g10 = g1_v[pl.ds(s, LANES)]
            b10 = b1_v[pl.ds(s, LANES)]
            r01 = plsc.load_gather(r0_v, [tvec1])
            g01 = plsc.load_gather(g0_v, [tvec1])
            b01 = plsc.load_gather(b0_v, [tvec1])
            r11 = plsc.load_gather(r1_v, [tvec1])
            g11 = plsc.load_gather(g1_v, [tvec1])
            b11 = plsc.load_gather(b1_v, [tvec1])
            I = plsc.PackFormat.INTERLEAVED
            ws = (plsc.pack(r00, g00, format=I), plsc.pack(b00, r01, format=I),
                  plsc.pack(g01, b01, format=I), plsc.pack(r10, g10, format=I),
                  plsc.pack(b10, r11, format=I), plsc.pack(g11, b11, format=I))
            for k in range(6):
                plsc.store_scatter(tab_v, [tvec, cw[k]],
                                   plsc.bitcast(ws[k], jnp.float32))
            return 0
        lax.fori_loop(0, W_RES // LANES, t_body, 0)

        pltpu.sync_copy(tab_v, tab_hbm.at[pl.ds(o0, W_RES)])
        return 0

    lax.fori_loop(0, H_RES // NW, chunk_body, 0)


_table_stage = functools.partial(
    pl.kernel,
    out_type=jax.ShapeDtypeStruct((HW, 8), jnp.float32),
    mesh=plsc.VectorSubcoreMesh(core_axis_name="c", subcore_axis_name="s"),
    compiler_params=pltpu.CompilerParams(needs_layout_passes=False,
                                         use_tc_tiling_on_sc=False),
    scratch_types=[
        pltpu.VMEM((W_RES + 8,), jnp.float32),
        pltpu.VMEM((W_RES + 8,), jnp.float32),
        pltpu.VMEM((W_RES + 8,), jnp.float32),
        pltpu.VMEM((W_RES + 8,), jnp.float32),
        pltpu.VMEM((W_RES + 8,), jnp.float32),
        pltpu.VMEM((W_RES + 8,), jnp.float32),
        pltpu.VMEM((W_RES, 8), jnp.float32),
        pltpu.SemaphoreType.DMA,
    ],
)(_table_body)


NCHUNKS = DW // CHUNK


def _sc_body(pk_hbm, fx_hbm, fy_hbm, tab_hbm,
             or_hbm, og_hbm, ob_hbm,
             pk_v, fx_v, fy_v, rows_v, o0_v, o1_v, o2_v,
             semi0, semi1, semg0, semg1, semo0, semo1):
    wid = lax.axis_index("s") * NUM_CORES + lax.axis_index("c")
    lanes = lax.broadcasted_iota(jnp.int32, (LANES,), 0)
    cw = [jnp.full((LANES,), k, jnp.int32) for k in range(6)]
    base = wid * DW
    semi = (semi0, semi1)
    semg = (semg0, semg1)
    semo = (semo0, semo1)

    def in_copies(c):
        bs = c % 2
        off = base + c * CHUNK
        return [pltpu.make_async_copy(h.at[pl.ds(off, CHUNK)],
                                      v.at[bs], semi[bs])
                for h, v in ((pk_hbm, pk_v), (fx_hbm, fx_v), (fy_hbm, fy_v))]

    def gather_copy(c):
        bs = c % 2
        return pltpu.make_async_copy(tab_hbm.at[pk_v.at[bs]],
                                     rows_v.at[bs], semg[bs])

    def out_copies(c):
        bs = c % 2
        off = base + c * CHUNK
        return [pltpu.make_async_copy(v.at[bs], h.at[pl.ds(off, CHUNK)],
                                      semo[bs])
                for v, h in ((o0_v, or_hbm), (o1_v, og_hbm), (o2_v, ob_hbm))]

    def s1(c):
        for cp in in_copies(c):
            cp.start()

    def s2(c):
        for cp in in_copies(c):
            cp.wait()
        gather_copy(c).start()

    def s3(c):
        bs = c % 2
        gather_copy(c).wait()
        if c >= 2:
            for cp in out_copies(c - 2):
                cp.wait()

        def c_body(j, _):
            s = j * LANES
            dvec = lanes + s
            fx16 = fx_v[bs, pl.ds(s, LANES)]
            fy16 = fy_v[bs, pl.ds(s, LANES)]
            gx = 1.0 - fx16
            gy = 1.0 - fy16
            w00 = gx * gy
            w01 = fx16 * gy
            w10 = gx * fy16
            w11 = fx16 * fy16
            I = plsc.PackFormat.INTERLEAVED
            rbs = rows_v.at[bs]
            wv = [plsc.bitcast(plsc.load_gather(rbs, [dvec, cw[k]]),
                               jnp.bfloat16) for k in range(6)]
            r00, g00 = plsc.unpack(wv[0], format=I)
            b00, r01 = plsc.unpack(wv[1], format=I)
            g01, b01 = plsc.unpack(wv[2], format=I)
            r10, g10 = plsc.unpack(wv[3], format=I)
            b10, r11 = plsc.unpack(wv[4], format=I)
            g11, b11 = plsc.unpack(wv[5], format=I)
            o0_v[bs, pl.ds(s, LANES)] = jnp.exp(
                w00 * r00 + w01 * r01 + w10 * r10 + w11 * r11)
            o1_v[bs, pl.ds(s, LANES)] = jnp.exp(
                w00 * g00 + w01 * g01 + w10 * g10 + w11 * g11)
            o2_v[bs, pl.ds(s, LANES)] = jnp.exp(
                w00 * b00 + w01 * b01 + w10 * b10 + w11 * b11)
            return 0
        lax.fori_loop(0, CHUNK // LANES, c_body, 0)

        for cp in out_copies(c):
            cp.start()

    s1(0)
    s1(1)
    s2(0)
    for c in range(NCHUNKS):
        if c + 1 < NCHUNKS:
            s2(c + 1)
        s3(c)
        if c + 2 < NCHUNKS:
            s1(c + 2)
    for cp in out_copies(NCHUNKS - 2):
        cp.wait()
    for cp in out_copies(NCHUNKS - 1):
        cp.wait()


_sc_stage = functools.partial(
    pl.kernel,
    out_type=[jax.ShapeDtypeStruct((N_TOTAL,), jnp.float32)] * 3,
    mesh=plsc.VectorSubcoreMesh(core_axis_name="c", subcore_axis_name="s"),
    compiler_params=pltpu.CompilerParams(needs_layout_passes=False,
                                         use_tc_tiling_on_sc=False),
    scratch_types=[
        pltpu.VMEM((2, CHUNK), jnp.int32),
        pltpu.VMEM((2, CHUNK), jnp.float32),
        pltpu.VMEM((2, CHUNK), jnp.float32),
        pltpu.VMEM((2, CHUNK, 8), jnp.float32),
        pltpu.VMEM((2, CHUNK), jnp.float32),
        pltpu.VMEM((2, CHUNK), jnp.float32),
        pltpu.VMEM((2, CHUNK), jnp.float32),
        pltpu.SemaphoreType.DMA,
        pltpu.SemaphoreType.DMA,
        pltpu.SemaphoreType.DMA,
        pltpu.SemaphoreType.DMA,
        pltpu.SemaphoreType.DMA,
        pltpu.SemaphoreType.DMA,
    ],
)(_sc_body)


def kernel(l, base):
    lx = l[:, 0].reshape(ROWS, COLS)
    ly = l[:, 1].reshape(ROWS, COLS)
    lz = l[:, 2].reshape(ROWS, COLS)
    pk, fx, fy = _uv_stage(lx, ly, lz)
    p0, p1, p2 = _detile_stage(base[:, :, 0], base[:, :, 1], base[:, :, 2])
    table = _table_stage(p0.reshape(HW), p1.reshape(HW), p2.reshape(HW))
    outr, outg, outb = _sc_stage(pk.reshape(N_TOTAL), fx.reshape(N_TOTAL),
                                 fy.reshape(N_TOTAL), table)
    return jnp.stack([outr, outg, outb], axis=1)


# speedup vs baseline: 28.7477x; 1.0936x over previous
import functools

import jax
import jax.numpy as jnp
from jax import lax
from jax.experimental import pallas as pl
from jax.experimental.pallas import tpu as pltpu
from jax.experimental.pallas import tpu_sc as plsc

H_RES, W_RES = 1024, 2048
HW = H_RES * W_RES
N_TOTAL = 2097152
INV_2PI = 0.15915494309189535
INV_PI = 0.3183098861837907

ROWS, COLS = 16384, 128
TC_BLOCK_ROWS = 2048

NUM_CORES, NUM_SUBCORES = 2, 16
NW = NUM_CORES * NUM_SUBCORES
DW = N_TOTAL // NW
CHUNK = 4096
TW = HW // NW
TCHUNK = 2048
LANES = 16


_ATAN_C = (0.9999998864165668, -0.33332597213024234, 0.1998590967770574,
           -0.1416124947328797, 0.10499020128168486, -0.07235009357159948,
           0.03978298699370322, -0.014402436802774879, 0.002456994955313291)
_PI = 3.141592653589793
_HALF_PI = 1.5707963267948966


def _atan01(r):
    r2 = r * r
    p = jnp.float32(_ATAN_C[-1])
    for c in reversed(_ATAN_C[:-1]):
        p = p * r2 + jnp.float32(c)
    return p * r


def _atan2(a, b):
    aa = jnp.abs(a)
    ab = jnp.abs(b)
    m = jnp.minimum(aa, ab)
    big = jnp.maximum(aa, ab)
    r = m / jnp.maximum(big, 1e-30)
    t = _atan01(r)
    t = jnp.where(aa > ab, _HALF_PI - t, t)
    t = jnp.where(b < 0.0, _PI - t, t)
    return jnp.where(a < 0.0, -t, t)


def _uv_body(lx_ref, ly_ref, lz_ref, pk_ref, fx_ref, fy_ref):
    x = lx_ref[...]
    y = ly_ref[...]
    z = lz_ref[...]
    u = _atan2(x, -z)
    u = u * INV_2PI + 0.5
    yc = jnp.clip(y, -1.0 + 1e-6, 1.0 - 1e-6)
    s = jnp.sqrt(jnp.maximum((1.0 - yc) * (1.0 + yc), 0.0))
    v = _atan2(s, yc) * INV_PI
    u = jnp.clip(u, 0.0, 1.0)
    v = jnp.clip(v, 0.0, 1.0)
    px = u * W_RES - 0.5
    py = v * H_RES - 0.5
    px0 = jnp.floor(px)
    py0 = jnp.floor(py)
    fx_ref[...] = px - px0
    fy_ref[...] = py - py0
    xi = px0.astype(jnp.int32) & (W_RES - 1)
    yi = py0.astype(jnp.int32) & (H_RES - 1)
    pk_ref[...] = (yi << 11) | xi


def _uv_stage(lx, ly, lz):
    grid = ROWS // TC_BLOCK_ROWS
    spec = pl.BlockSpec((TC_BLOCK_ROWS, COLS), lambda i: (i, 0))
    return pl.pallas_call(
        _uv_body,
        grid=(grid,),
        in_specs=[spec, spec, spec],
        out_specs=[spec, spec, spec],
        out_shape=[
            jax.ShapeDtypeStruct((ROWS, COLS), jnp.int32),
            jax.ShapeDtypeStruct((ROWS, COLS), jnp.float32),
            jax.ShapeDtypeStruct((ROWS, COLS), jnp.float32),
        ],
    )(lx, ly, lz)


def _detile_body(p0_ref, p1_ref, p2_ref, o0_ref, o1_ref, o2_ref):
    o0_ref[:, 0, 0, :] = p0_ref[...]
    o1_ref[:, 0, 0, :] = p1_ref[...]
    o2_ref[:, 0, 0, :] = p2_ref[...]


def _detile_stage(p0, p1, p2):
    in_spec = pl.BlockSpec((1024, 128), lambda i, j: (i, j))
    out_spec = pl.BlockSpec((1024, 1, 1, 128), lambda i, j: (i, j, 0, 0))
    return pl.pallas_call(
        _detile_body,
        grid=(H_RES // 1024, W_RES // 128),
        in_specs=[in_spec, in_spec, in_spec],
        out_specs=[out_spec, out_spec, out_spec],
        out_shape=[jax.ShapeDtypeStruct((H_RES, W_RES // 128, 1, 128),
                                        jnp.float32)] * 3,
    )(p0, p1, p2)


ROWS_PER_W = H_RES // NW


def _table_body(pr_hbm, pg_hbm, pb_hbm, tab_hbm,
                r_v, g_v, b_v, tab_v,
                semp0, semp1, semp2, semt0, semt1):
    wid = lax.axis_index("s") * NUM_CORES + lax.axis_index("c")
    lanes = lax.broadcasted_iota(jnp.int32, (LANES,), 0)
    cw = [jnp.full((LANES,), k, jnp.int32) for k in range(6)]
    wbase = wid * ROWS_PER_W
    semp = (semp0, semp1, semp2)
    semt = (semt0, semt1)

    def row_copies(r, sset):
        o = (jnp.bitwise_and(wbase + r, H_RES - 1)) * W_RES
        cps = []
        for src_hbm, dst in ((pr_hbm, r_v), (pg_hbm, g_v), (pb_hbm, b_v)):
            cps.append(pltpu.make_async_copy(
                src_hbm.at[pl.ds(o, W_RES)],
                dst.at[sset].at[pl.ds(0, W_RES)], semp[sset]))
            cps.append(pltpu.make_async_copy(
                src_hbm.at[pl.ds(o, 8)],
                dst.at[sset].at[pl.ds(W_RES, 8)], semp[sset]))
        return cps

    def tab_copy(ci):
        bs = ci % 2
        o = (wbase + ci) * W_RES
        return pltpu.make_async_copy(tab_v.at[bs],
                                     tab_hbm.at[pl.ds(o, W_RES)], semt[bs])

    for cp in row_copies(0, 0):
        cp.start()
    for cp in row_copies(1, 1):
        cp.start()
    for cp in row_copies(0, 0):
        cp.wait()

    for ci in range(ROWS_PER_W):
        cur, nxt = ci % 3, (ci + 1) % 3
        bs = ci % 2
        if ci + 2 < ROWS_PER_W + 1:
            for cp in row_copies(ci + 2, (ci + 2) % 3):
                cp.start()
        for cp in row_copies(ci + 1, nxt):
            cp.wait()
        if ci >= 2:
            tab_copy(ci - 2).wait()

        r0c, g0c, b0c = r_v.at[cur], g_v.at[cur], b_v.at[cur]
        r1c, g1c, b1c = r_v.at[nxt], g_v.at[nxt], b_v.at[nxt]
        tb = tab_v.at[bs]

        def t_body(j, _):
            s = j * LANES
            tvec = lanes + s
            tvec1 = tvec + 1
            r00 = r0c[pl.ds(s, LANES)]
            g00 = g0c[pl.ds(s, LANES)]
            b00 = b0c[pl.ds(s, LANES)]
            r10 = r1c[pl.ds(s, LANES)]
            g10 = g1c[pl.ds(s, LANES)]
            b10 = b1c[pl.ds(s, LANES)]
            r01 = plsc.load_gather(r0c, [tvec1])
            g01 = plsc.load_gather(g0c, [tvec1])
            b01 = plsc.load_gather(b0c, [tvec1])
            r11 = plsc.load_gather(r1c, [tvec1])
            g11 = plsc.load_gather(g1c, [tvec1])
            b11 = plsc.load_gather(b1c, [tvec1])
            I = plsc.PackFormat.INTERLEAVED
            ws = (plsc.pack(r00, g00, format=I), plsc.pack(b00, r01, format=I),
                  plsc.pack(g01, b01, format=I), plsc.pack(r10, g10, format=I),
                  plsc.pack(b10, r11, format=I), plsc.pack(g11, b11, format=I))
            for k in range(6):
                plsc.store_scatter(tb, [tvec, cw[k]],
                                   plsc.bitcast(ws[k], jnp.float32))
            return 0
        lax.fori_loop(0, W_RES // LANES, t_body, 0)

        tab_copy(ci).start()

    tab_copy(ROWS_PER_W - 2).wait()
    tab_copy(ROWS_PER_W - 1).wait()


_table_stage = functools.partial(
    pl.kernel,
    out_type=jax.ShapeDtypeStruct((HW, 8), jnp.float32),
    mesh=plsc.VectorSubcoreMesh(core_axis_name="c", subcore_axis_name="s"),
    compiler_params=pltpu.CompilerParams(needs_layout_passes=False,
                                         use_tc_tiling_on_sc=False),
    scratch_types=[
        pltpu.VMEM((3, W_RES + 8), jnp.float32),
        pltpu.VMEM((3, W_RES + 8), jnp.float32),
        pltpu.VMEM((3, W_RES + 8), jnp.float32),
        pltpu.VMEM((2, W_RES, 8), jnp.float32),
        pltpu.SemaphoreType.DMA,
        pltpu.SemaphoreType.DMA,
        pltpu.SemaphoreType.DMA,
        pltpu.SemaphoreType.DMA,
        pltpu.SemaphoreType.DMA,
    ],
)(_table_body)


NCHUNKS = DW // CHUNK


def _sc_body(pk_hbm, fx_hbm, fy_hbm, tab_hbm,
             or_hbm, og_hbm, ob_hbm,
             pk_v, fx_v, fy_v, rows_v, o0_v, o1_v, o2_v,
             semi0, semi1, semg0, semg1, semo0, semo1):
    wid = lax.axis_index("s") * NUM_CORES + lax.axis_index("c")
    lanes = lax.broadcasted_iota(jnp.int32, (LANES,), 0)
    cw = [jnp.full((LANES,), k, jnp.int32) for k in range(6)]
    base = wid * DW
    semi = (semi0, semi1)
    semg = (semg0, semg1)
    semo = (semo0, semo1)

    def in_copies(c):
        bs = c % 2
        off = base + c * CHUNK
        return [pltpu.make_async_copy(h.at[pl.ds(off, CHUNK)],
                                      v.at[bs], semi[bs])
                for h, v in ((pk_hbm, pk_v), (fx_hbm, fx_v), (fy_hbm, fy_v))]

    def gather_copy(c):
        bs = c % 2
        return pltpu.make_async_copy(tab_hbm.at[pk_v.at[bs]],
                                     rows_v.at[bs], semg[bs])

    def out_copies(c):
        bs = c % 2
        off = base + c * CHUNK
        return [pltpu.make_async_copy(v.at[bs], h.at[pl.ds(off, CHUNK)],
                                      semo[bs])
                for v, h in ((o0_v, or_hbm), (o1_v, og_hbm), (o2_v, ob_hbm))]

    def s1(c):
        for cp in in_copies(c):
            cp.start()

    def s2(c):
        for cp in in_copies(c):
            cp.wait()
        gather_copy(c).start()

    def s3(c):
        bs = c % 2
        gather_copy(c).wait()
        if c >= 2:
            for cp in out_copies(c - 2):
                cp.wait()

        def c_body(j, _):
            s = j * LANES
            dvec = lanes + s
            fx16 = fx_v[bs, pl.ds(s, LANES)]
            fy16 = fy_v[bs, pl.ds(s, LANES)]
            gx = 1.0 - fx16
            gy = 1.0 - fy16
            w00 = gx * gy
            w01 = fx16 * gy
            w10 = gx * fy16
            w11 = fx16 * fy16
            I = plsc.PackFormat.INTERLEAVED
            rbs = rows_v.at[bs]
            wv = [plsc.bitcast(plsc.load_gather(rbs, [dvec, cw[k]]),
                               jnp.bfloat16) for k in range(6)]
            r00, g00 = plsc.unpack(wv[0], format=I)
            b00, r01 = plsc.unpack(wv[1], format=I)
            g01, b01 = plsc.unpack(wv[2], format=I)
            r10, g10 = plsc.unpack(wv[3], format=I)
            b10, r11 = plsc.unpack(wv[4], format=I)
            g11, b11 = plsc.unpack(wv[5], format=I)
            o0_v[bs, pl.ds(s, LANES)] = jnp.exp(
                w00 * r00 + w01 * r01 + w10 * r10 + w11 * r11)
            o1_v[bs, pl.ds(s, LANES)] = jnp.exp(
                w00 * g00 + w01 * g01 + w10 * g10 + w11 * g11)
            o2_v[bs, pl.ds(s, LANES)] = jnp.exp(
                w00 * b00 + w01 * b01 + w10 * b10 + w11 * b11)
            return 0
        lax.fori_loop(0, CHUNK // LANES, c_body, 0)

        for cp in out_copies(c):
            cp.start()

    s1(0)
    s1(1)
    s2(0)
    for c in range(NCHUNKS):
        if c + 1 < NCHUNKS:
            s2(c + 1)
        s3(c)
        if c + 2 < NCHUNKS:
            s1(c + 2)
    for cp in out_copies(NCHUNKS - 2):
        cp.wait()
    for cp in out_copies(NCHUNKS - 1):
        cp.wait()


_sc_stage = functools.partial(
    pl.kernel,
    out_type=[jax.ShapeDtypeStruct((N_TOTAL,), jnp.float32)] * 3,
    mesh=plsc.VectorSubcoreMesh(core_axis_name="c", subcore_axis_name="s"),
    compiler_params=pltpu.CompilerParams(needs_layout_passes=False,
                                         use_tc_tiling_on_sc=False),
    scratch_types=[
        pltpu.VMEM((2, CHUNK), jnp.int32),
        pltpu.VMEM((2, CHUNK), jnp.float32),
        pltpu.VMEM((2, CHUNK), jnp.float32),
        pltpu.VMEM((2, CHUNK, 8), jnp.float32),
        pltpu.VMEM((2, CHUNK), jnp.float32),
        pltpu.VMEM((2, CHUNK), jnp.float32),
        pltpu.VMEM((2, CHUNK), jnp.float32),
        pltpu.SemaphoreType.DMA,
        pltpu.SemaphoreType.DMA,
        pltpu.SemaphoreType.DMA,
        pltpu.SemaphoreType.DMA,
        pltpu.SemaphoreType.DMA,
        pltpu.SemaphoreType.DMA,
    ],
)(_sc_body)


def kernel(l, base):
    lx = l[:, 0].reshape(ROWS, COLS)
    ly = l[:, 1].reshape(ROWS, COLS)
    lz = l[:, 2].reshape(ROWS, COLS)
    pk, fx, fy = _uv_stage(lx, ly, lz)
    p0, p1, p2 = _detile_stage(base[:, :, 0], base[:, :, 1], base[:, :, 2])
    table = _table_stage(p0.reshape(HW), p1.reshape(HW), p2.reshape(HW))
    outr, outg, outb = _sc_stage(pk.reshape(N_TOTAL), fx.reshape(N_TOTAL),
                                 fy.reshape(N_TOTAL), table)
    return jnp.stack([outr, outg, outb], axis=1)


# speedup vs baseline: 29.1980x; 1.0157x over previous
import functools

import jax
import jax.numpy as jnp
from jax import lax
from jax.experimental import pallas as pl
from jax.experimental.pallas import tpu as pltpu
from jax.experimental.pallas import tpu_sc as plsc

H_RES, W_RES = 1024, 2048
HW = H_RES * W_RES
N_TOTAL = 2097152
INV_2PI = 0.15915494309189535
INV_PI = 0.3183098861837907

ROWS, COLS = 16384, 128
TC_BLOCK_ROWS = 2048

NUM_CORES, NUM_SUBCORES = 2, 16
NW = NUM_CORES * NUM_SUBCORES
DW = N_TOTAL // NW
CHUNK = 4096
TW = HW // NW
TCHUNK = 2048
LANES = 16


_ATAN_C = (0.9999998864165668, -0.33332597213024234, 0.1998590967770574,
           -0.1416124947328797, 0.10499020128168486, -0.07235009357159948,
           0.03978298699370322, -0.014402436802774879, 0.002456994955313291)
_PI = 3.141592653589793
_HALF_PI = 1.5707963267948966


def _atan01(r):
    r2 = r * r
    p = jnp.float32(_ATAN_C[-1])
    for c in reversed(_ATAN_C[:-1]):
        p = p * r2 + jnp.float32(c)
    return p * r


def _atan2(a, b):
    aa = jnp.abs(a)
    ab = jnp.abs(b)
    m = jnp.minimum(aa, ab)
    big = jnp.maximum(aa, ab)
    r = m / jnp.maximum(big, 1e-30)
    t = _atan01(r)
    t = jnp.where(aa > ab, _HALF_PI - t, t)
    t = jnp.where(b < 0.0, _PI - t, t)
    return jnp.where(a < 0.0, -t, t)


def _uv_body(lx_ref, ly_ref, lz_ref, pk_ref, fx_ref, fy_ref):
    x = lx_ref[...]
    y = ly_ref[...]
    z = lz_ref[...]
    u = _atan2(x, -z)
    u = u * INV_2PI + 0.5
    yc = jnp.clip(y, -1.0 + 1e-6, 1.0 - 1e-6)
    s = jnp.sqrt(jnp.maximum((1.0 - yc) * (1.0 + yc), 0.0))
    v = _atan2(s, yc) * INV_PI
    u = jnp.clip(u, 0.0, 1.0)
    v = jnp.clip(v, 0.0, 1.0)
    px = u * W_RES - 0.5
    py = v * H_RES - 0.5
    px0 = jnp.floor(px)
    py0 = jnp.floor(py)
    fx_ref[...] = px - px0
    fy_ref[...] = py - py0
    xi = px0.astype(jnp.int32) & (W_RES - 1)
    yi = py0.astype(jnp.int32) & (H_RES - 1)
    pk_ref[...] = (yi << 11) | xi


def _uv_stage(lx, ly, lz):
    grid = ROWS // TC_BLOCK_ROWS
    spec = pl.BlockSpec((TC_BLOCK_ROWS, COLS), lambda i: (i, 0))
    return pl.pallas_call(
        _uv_body,
        grid=(grid,),
        in_specs=[spec, spec, spec],
        out_specs=[spec, spec, spec],
        out_shape=[
            jax.ShapeDtypeStruct((ROWS, COLS), jnp.int32),
            jax.ShapeDtypeStruct((ROWS, COLS), jnp.float32),
            jax.ShapeDtypeStruct((ROWS, COLS), jnp.float32),
        ],
    )(lx, ly, lz)


def _detile_body(p0_ref, p1_ref, p2_ref, o0_ref, o1_ref, o2_ref):
    o0_ref[:, 0, 0, :] = p0_ref[...]
    o1_ref[:, 0, 0, :] = p1_ref[...]
    o2_ref[:, 0, 0, :] = p2_ref[...]


def _detile_stage(p0, p1, p2):
    in_spec = pl.BlockSpec((1024, 128), lambda i, j: (i, j))
    out_spec = pl.BlockSpec((1024, 1, 1, 128), lambda i, j: (i, j, 0, 0))
    return pl.pallas_call(
        _detile_body,
        grid=(H_RES // 1024, W_RES // 128),
        in_specs=[in_spec, in_spec, in_spec],
        out_specs=[out_spec, out_spec, out_spec],
        out_shape=[jax.ShapeDtypeStruct((H_RES, W_RES // 128, 1, 128),
                                        jnp.float32)] * 3,
    )(p0, p1, p2)


ROWS_PER_W = H_RES // NW


def _table_body(pr_hbm, pg_hbm, pb_hbm, tab_hbm,
                r_v, g_v, b_v, tab_v,
                semp0, semp1, semp2, semt0, semt1):
    wid = lax.axis_index("s") * NUM_CORES + lax.axis_index("c")
    lanes = lax.broadcasted_iota(jnp.int32, (LANES,), 0)
    cw = [jnp.full((LANES,), k, jnp.int32) for k in range(6)]
    wbase = wid * ROWS_PER_W
    semp = (semp0, semp1, semp2)
    semt = (semt0, semt1)

    def row_copies(r, sset):
        o = (jnp.bitwise_and(wbase + r, H_RES - 1)) * W_RES
        cps = []
        for src_hbm, dst in ((pr_hbm, r_v), (pg_hbm, g_v), (pb_hbm, b_v)):
            cps.append(pltpu.make_async_copy(
                src_hbm.at[pl.ds(o, W_RES)],
                dst.at[sset].at[pl.ds(0, W_RES)], semp[sset]))
            cps.append(pltpu.make_async_copy(
                src_hbm.at[pl.ds(o, 8)],
                dst.at[sset].at[pl.ds(W_RES, 8)], semp[sset]))
        return cps

    def tab_copy(ci):
        bs = ci % 2
        o = (wbase + ci) * W_RES
        return pltpu.make_async_copy(tab_v.at[bs],
                                     tab_hbm.at[pl.ds(o, W_RES)], semt[bs])

    for cp in row_copies(0, 0):
        cp.start()
    for cp in row_copies(1, 1):
        cp.start()
    for cp in row_copies(0, 0):
        cp.wait()

    for ci in range(ROWS_PER_W):
        cur, nxt = ci % 3, (ci + 1) % 3
        bs = ci % 2
        if ci + 2 < ROWS_PER_W + 1:
            for cp in row_copies(ci + 2, (ci + 2) % 3):
                cp.start()
        for cp in row_copies(ci + 1, nxt):
            cp.wait()
        if ci >= 2:
            tab_copy(ci - 2).wait()

        r0c, g0c, b0c = r_v.at[cur], g_v.at[cur], b_v.at[cur]
        r1c, g1c, b1c = r_v.at[nxt], g_v.at[nxt], b_v.at[nxt]
        tb = tab_v.at[bs]

        def t_body(j, _):
            s = j * LANES
            tvec = lanes + s
            tvec1 = tvec + 1
            r00 = r0c[pl.ds(s, LANES)]
            g00 = g0c[pl.ds(s, LANES)]
            b00 = b0c[pl.ds(s, LANES)]
            r10 = r1c[pl.ds(s, LANES)]
            g10 = g1c[pl.ds(s, LANES)]
            b10 = b1c[pl.ds(s, LANES)]
            r01 = plsc.load_gather(r0c, [tvec1])
            g01 = plsc.load_gather(g0c, [tvec1])
            b01 = plsc.load_gather(b0c, [tvec1])
            r11 = plsc.load_gather(r1c, [tvec1])
            g11 = plsc.load_gather(g1c, [tvec1])
            b11 = plsc.load_gather(b1c, [tvec1])
            I = plsc.PackFormat.INTERLEAVED
            ws = (plsc.pack(r00, g00, format=I), plsc.pack(b00, r01, format=I),
                  plsc.pack(g01, b01, format=I), plsc.pack(r10, g10, format=I),
                  plsc.pack(b10, r11, format=I), plsc.pack(g11, b11, format=I))
            for k in range(6):
                plsc.store_scatter(tb, [tvec, cw[k]],
                                   plsc.bitcast(ws[k], jnp.float32))
            return 0
        lax.fori_loop(0, W_RES // LANES, t_body, 0)

        tab_copy(ci).start()

    tab_copy(ROWS_PER_W - 2).wait()
    tab_copy(ROWS_PER_W - 1).wait()


_table_stage = functools.partial(
    pl.kernel,
    out_type=jax.ShapeDtypeStruct((HW, 8), jnp.float32),
    mesh=plsc.VectorSubcoreMesh(core_axis_name="c", subcore_axis_name="s"),
    compiler_params=pltpu.CompilerParams(needs_layout_passes=False,
                                         use_tc_tiling_on_sc=False),
    scratch_types=[
        pltpu.VMEM((3, W_RES + 8), jnp.float32),
        pltpu.VMEM((3, W_RES + 8), jnp.float32),
        pltpu.VMEM((3, W_RES + 8), jnp.float32),
        pltpu.VMEM((2, W_RES, 8), jnp.float32),
        pltpu.SemaphoreType.DMA,
        pltpu.SemaphoreType.DMA,
        pltpu.SemaphoreType.DMA,
        pltpu.SemaphoreType.DMA,
        pltpu.SemaphoreType.DMA,
    ],
)(_table_body)


NCHUNKS = DW // CHUNK


def _sc_body(pk_hbm, fx_hbm, fy_hbm, tab_hbm,
             or_hbm, og_hbm, ob_hbm,
             pk_v, fx_v, fy_v, rows_v, o0_v, o1_v, o2_v,
             semi0, semi1, semg0, semg1, semo0, semo1):
    wid = lax.axis_index("s") * NUM_CORES + lax.axis_index("c")
    lanes = lax.broadcasted_iota(jnp.int32, (LANES,), 0)
    cw = [jnp.full((LANES,), k, jnp.int32) for k in range(6)]
    base = wid * DW
    semi = (semi0, semi1)
    semg = (semg0, semg1)
    semo = (semo0, semo1)

    def in_copies(c):
        bs = c % 2
        off = base + c * CHUNK
        return [pltpu.make_async_copy(h.at[pl.ds(off, CHUNK)],
                                      v.at[bs], semi[bs])
                for h, v in ((pk_hbm, pk_v), (fx_hbm, fx_v), (fy_hbm, fy_v))]

    def gather_copy(c):
        bs = c % 2
        return pltpu.make_async_copy(tab_hbm.at[pk_v.at[bs]],
                                     rows_v.at[bs], semg[bs])

    def out_copies(c):
        bs = c % 2
        off = base + c * CHUNK
        return [pltpu.make_async_copy(v.at[bs], h.at[pl.ds(off, CHUNK)],
                                      semo[bs])
                for v, h in ((o0_v, or_hbm), (o1_v, og_hbm), (o2_v, ob_hbm))]

    def s1(c):
        for cp in in_copies(c):
            cp.start()

    def s2(c):
        for cp in in_copies(c):
            cp.wait()
        gather_copy(c).start()

    def s3(c):
        bs = c % 2
        gather_copy(c).wait()
        if c >= 2:
            for cp in out_copies(c - 2):
                cp.wait()

        def c_half(s):
            dvec = lanes + s
            fx16 = fx_v[bs, pl.ds(s, LANES)]
            fy16 = fy_v[bs, pl.ds(s, LANES)]
            gx = 1.0 - fx16
            gy = 1.0 - fy16
            w00 = gx * gy
            w01 = fx16 * gy
            w10 = gx * fy16
            w11 = fx16 * fy16
            I = plsc.PackFormat.INTERLEAVED
            rbs = rows_v.at[bs]
            wv = [plsc.bitcast(plsc.load_gather(rbs, [dvec, cw[k]]),
                               jnp.bfloat16) for k in range(6)]
            r00, g00 = plsc.unpack(wv[0], format=I)
            b00, r01 = plsc.unpack(wv[1], format=I)
            g01, b01 = plsc.unpack(wv[2], format=I)
            r10, g10 = plsc.unpack(wv[3], format=I)
            b10, r11 = plsc.unpack(wv[4], format=I)
            g11, b11 = plsc.unpack(wv[5], format=I)
            o0_v[bs, pl.ds(s, LANES)] = jnp.exp(
                w00 * r00 + w01 * r01 + w10 * r10 + w11 * r11)
            o1_v[bs, pl.ds(s, LANES)] = jnp.exp(
                w00 * g00 + w01 * g01 + w10 * g10 + w11 * g11)
            o2_v[bs, pl.ds(s, LANES)] = jnp.exp(
                w00 * b00 + w01 * b01 + w10 * b10 + w11 * b11)

        def c_body(j, _):
            s = j * (2 * LANES)
            c_half(s)
            c_half(s + LANES)
            return 0
        lax.fori_loop(0, CHUNK // (2 * LANES), c_body, 0)

        for cp in out_copies(c):
            cp.start()

    s1(0)
    s1(1)
    s2(0)
    for c in range(NCHUNKS):
        if c + 1 < NCHUNKS:
            s2(c + 1)
        s3(c)
        if c + 2 < NCHUNKS:
            s1(c + 2)
    for cp in out_copies(NCHUNKS - 2):
        cp.wait()
    for cp in out_copies(NCHUNKS - 1):
        cp.wait()


_sc_stage = functools.partial(
    pl.kernel,
    out_type=[jax.ShapeDtypeStruct((N_TOTAL,), jnp.float32)] * 3,
    mesh=plsc.VectorSubcoreMesh(core_axis_name="c", subcore_axis_name="s"),
    compiler_params=pltpu.CompilerParams(needs_layout_passes=False,
                                         use_tc_tiling_on_sc=False),
    scratch_types=[
        pltpu.VMEM((2, CHUNK), jnp.int32),
        pltpu.VMEM((2, CHUNK), jnp.float32),
        pltpu.VMEM((2, CHUNK), jnp.float32),
        pltpu.VMEM((2, CHUNK, 8), jnp.float32),
        pltpu.VMEM((2, CHUNK), jnp.float32),
        pltpu.VMEM((2, CHUNK), jnp.float32),
        pltpu.VMEM((2, CHUNK), jnp.float32),
        pltpu.SemaphoreType.DMA,
        pltpu.SemaphoreType.DMA,
        pltpu.SemaphoreType.DMA,
        pltpu.SemaphoreType.DMA,
        pltpu.SemaphoreType.DMA,
        pltpu.SemaphoreType.DMA,
    ],
)(_sc_body)


def kernel(l, base):
    lx = l[:, 0].reshape(ROWS, COLS)
    ly = l[:, 1].reshape(ROWS, COLS)
    lz = l[:, 2].reshape(ROWS, COLS)
    pk, fx, fy = _uv_stage(lx, ly, lz)
    p0, p1, p2 = _detile_stage(base[:, :, 0], base[:, :, 1], base[:, :, 2])
    table = _table_stage(p0.reshape(HW), p1.reshape(HW), p2.reshape(HW))
    outr, outg, outb = _sc_stage(pk.reshape(N_TOTAL), fx.reshape(N_TOTAL),
                                 fy.reshape(N_TOTAL), table)
    return jnp.stack([outr, outg, outb], axis=1)
